# baseline (speedup 1.0000x reference)
import functools

import jax
import jax.numpy as jnp
from jax import lax
from jax.experimental import pallas as pl
from jax.experimental.pallas import tpu as pltpu
from jax.experimental.pallas import tpu_sc as plsc

_NW = 32
_C = 4
_NB = 2


def _matmul(x, w, np_rows, bm):
    n, k = x.shape
    m = w.shape[1]

    def body(x_ref, w_ref, o_ref):
        o_ref[...] = jnp.dot(x_ref[...], w_ref[...],
                             preferred_element_type=jnp.float32)

    return pl.pallas_call(
        body,
        grid=(np_rows // bm,),
        in_specs=[pl.BlockSpec((bm, k), lambda i: (i, 0)),
                  pl.BlockSpec((k, m), lambda i: (0, 0))],
        out_specs=pl.BlockSpec((bm, m), lambda i: (i, 0)),
        out_shape=jax.ShapeDtypeStruct((np_rows, m), jnp.float32),
    )(x, w)


def _gat_sc_layer(N, NP, D, nheads, F, GW, apply_elu):
    npw = NP // _NW
    nchunks = npw // _C
    E = _C * D
    outw = nheads * F
    scol0 = nheads * F
    nacc = F // 16
    idx_lim = N * D - E
    mesh = plsc.VectorSubcoreMesh(core_axis_name="c", subcore_axis_name="s")

    @functools.partial(
        pl.kernel,
        mesh=mesh,
        compiler_params=pltpu.CompilerParams(use_tc_tiling_on_sc=False,
                                             needs_layout_passes=False),
        out_type=jax.ShapeDtypeStruct((NP, outw), jnp.float32),
        scratch_types=[
            *[pltpu.VMEM((E,), jnp.int32) for _ in range(_NB)],
            *[pltpu.VMEM((E, GW), jnp.float32) for _ in range(_NB)],
            *[pltpu.VMEM((_C, outw), jnp.float32) for _ in range(2)],
            pltpu.VMEM_SHARED((NP, GW), jnp.float32),
            *[pltpu.SemaphoreType.DMA for _ in range(_NB)],
            *[pltpu.SemaphoreType.DMA for _ in range(_NB)],
            *[pltpu.SemaphoreType.DMA for _ in range(2)],
        ],
    )
    def k(adjf, g, out, *scr):
        idx_vs = list(scr[0:_NB])
        rows_vs = list(scr[_NB:2 * _NB])
        o_vs = list(scr[2 * _NB:2 * _NB + 2])
        gs = scr[2 * _NB + 2]
        gsems = list(scr[2 * _NB + 3:3 * _NB + 3])
        isems = list(scr[3 * _NB + 3:4 * _NB + 3])
        osems = list(scr[4 * _NB + 3:4 * _NB + 5])
        sid = lax.axis_index("s")
        wid = sid * 2 + lax.axis_index("c")
        base = wid * npw

        rpt = NP // 16
        pltpu.sync_copy(g.at[pl.ds(sid * rpt, rpt)], gs.at[pl.ds(sid * rpt, rpt)])
        plsc.subcore_barrier()

        def start_idx(c, b):
            start = (base + c * _C) * D
            pltpu.make_async_copy(adjf.at[pl.ds(start, E)], idx_vs[b],
                                  isems[b]).start()

        def wait_idx(b):
            pltpu.make_async_copy(adjf.at[pl.ds(0, E)], idx_vs[b],
                                  isems[b]).wait()

        def start_gather(b):
            pltpu.make_async_copy(gs.at[idx_vs[b]], rows_vs[b], gsems[b]).start()

        def wait_gather(b):
            pltpu.make_async_copy(gs.at[idx_vs[b]], rows_vs[b], gsems[b]).wait()

        def wait_out(ob):
            pltpu.make_async_copy(o_vs[ob], out.at[pl.ds(base, _C)],
                                  osems[ob]).wait()

        def compute(c, rows_v, ob):
            nb = base + c * _C
            o_v = o_vs[ob]

            def node_body(i2, carry2):
                pairs = [(2 * i2, 2 * i2 * D), (2 * i2 + 1, (2 * i2 + 1) * D)]
                for i, r0_ in pairs:
                  stv = rows_v[r0_, pl.ds(scol0, 16)]
                  for h in range(nheads):
                    scol = scol0 + 2 * h
                    it = lax.iota(jnp.int32, 16)
                    cs = jnp.full((16,), scol, jnp.int32)
                    s0 = plsc.load_gather(rows_v, [r0_ + it, cs])
                    s1 = plsc.load_gather(rows_v, [r0_ + 16 + it, cs])
                    t = jnp.broadcast_to(stv[2 * h + 1], (16,))
                    e0 = s0 + t
                    e1 = s1 + t
                    e0 = jnp.maximum(e0, 0.2 * e0)
                    e1 = jnp.maximum(e1, 0.2 * e1)
                    p0 = jnp.exp(e0)
                    p1 = jnp.exp(e1)
                    z = jnp.sum(p0) + jnp.sum(p1)
                    zrv = 1.0 / jnp.broadcast_to(z, (16,))
                    accs = [[jnp.zeros((16,), jnp.float32) for _ in range(4)]
                            for _ in range(nacc)]
                    for d in range(D):
                        pd = jnp.broadcast_to((p0 if d < 16 else p1)[d % 16], (16,))
                        for fg in range(nacc):
                            col = h * F + fg * 16
                            accs[fg][d % 4] = accs[fg][d % 4] + pd * rows_v[r0_ + d, pl.ds(col, 16)]
                    for fg in range(nacc):
                        a4 = accs[fg]
                        acc = ((a4[0] + a4[1]) + (a4[2] + a4[3])) * zrv
                        if apply_elu:
                            acc = jnp.where(acc > 0.0, acc, jnp.exp(acc) - 1.0)
                        o_v[i, pl.ds(h * F + fg * 16, 16)] = acc
                return carry2

            lax.fori_loop(0, _C // 2, node_body, 0)
            pltpu.make_async_copy(o_v, out.at[pl.ds(nb, _C)], osems[ob]).start()

        for c in range(_NB):
            start_idx(c, c)
        for b in range(_NB - 1):
            wait_idx(b)
            start_gather(b)

        def quad_body(q, carry):
            c0 = q * _NB
            for b in range(_NB):
                c = c0 + b
                wait_gather(b)

                @pl.when(c + _NB < nchunks)
                def _():
                    start_idx(c + _NB, b)

                @pl.when(c + _NB - 1 < nchunks)
                def _():
                    wait_idx((b + _NB - 1) % _NB)
                    start_gather((b + _NB - 1) % _NB)

                if b >= 2:
                    wait_out(b % 2)
                else:
                    @pl.when(q > 0)
                    def _():
                        wait_out(b % 2)

                compute(c, rows_vs[b], b % 2)

            return carry

        lax.fori_loop(0, nchunks // _NB, quad_body, 0)
        wait_out(0)
        wait_out(1)

    return k


def _elu_logsoftmax(zin, n_valid):
    NPl, cls = zin.shape

    def body(z_ref, o_ref):
        zz = z_ref[...]
        x = jnp.where(zz > 0.0, zz, jnp.exp(zz) - 1.0)
        valid = lax.broadcasted_iota(jnp.int32, (NPl, cls), 0) < n_valid
        xm = jnp.where(valid, x, -jnp.inf)
        mx = jnp.max(xm, axis=0, keepdims=True)
        se = jnp.sum(jnp.exp(xm - mx), axis=0, keepdims=True)
        o_ref[...] = x - (mx + jnp.log(se))

    return pl.pallas_call(
        body,
        out_shape=jax.ShapeDtypeStruct((NPl, cls), jnp.float32),
    )(zin)


def kernel(embedding, adj, W_heads, a_heads, W_out, a_out):
    bs, N, nfeat = embedding.shape
    nheads, _, nhid = W_heads.shape
    D = adj.shape[2]
    nclass = W_out.shape[1]
    NP = -(-N // 1024) * 1024

    x = embedding.reshape(N, nfeat)
    xp = jnp.pad(x, ((0, NP - N), (0, 0)))
    adjf = jnp.pad(adj.reshape(N, D), ((0, NP - N), (0, 0))).reshape(NP * D)

    Wc = jnp.swapaxes(W_heads, 0, 1).reshape(nfeat, nheads * nhid)
    a1 = a_heads[:, :nhid, 0]
    a2 = a_heads[:, nhid:, 0]
    sW = jnp.einsum('hfk,hk->fh', W_heads, a1)
    tW = jnp.einsum('hfk,hk->fh', W_heads, a2)
    stW = jnp.stack([sW, tW], axis=2).reshape(nfeat, 2 * nheads)
    GW1 = 144
    M1 = jnp.concatenate(
        [Wc, stW,
         jnp.zeros((nfeat, GW1 - nheads * nhid - 2 * nheads), jnp.float32)],
        axis=1)
    G1 = _matmul(xp, M1, NP, 2048)

    l1 = _gat_sc_layer(N, NP, D, nheads, nhid, GW1, True)
    x1 = l1(adjf, G1)

    GW2 = 48
    M2 = jnp.concatenate(
        [W_out, W_out @ a_out[:nclass], W_out @ a_out[nclass:],
         jnp.zeros((nheads * nhid, GW2 - nclass - 2), jnp.float32)],
        axis=1)
    G2 = _matmul(x1, M2, NP, 2048)

    l2 = _gat_sc_layer(N, NP, D, 1, nclass, GW2, False)
    z = l2(adjf, G2)

    out = _elu_logsoftmax(z, N)
    return out[:N].reshape(bs, N, nclass)


# speedup vs baseline: 1.1023x; 1.1023x over previous
import functools

import jax
import jax.numpy as jnp
from jax import lax
from jax.experimental import pallas as pl
from jax.experimental.pallas import tpu as pltpu
from jax.experimental.pallas import tpu_sc as plsc

_NW = 32
_C = 4
_NB = 2


def _matmul(x, w, np_rows, bm):
    n, k = x.shape
    m = w.shape[1]

    def body(x_ref, w_ref, o_ref):
        o_ref[...] = jnp.dot(x_ref[...], w_ref[...],
                             preferred_element_type=jnp.float32)

    return pl.pallas_call(
        body,
        grid=(np_rows // bm,),
        in_specs=[pl.BlockSpec((bm, k), lambda i: (i, 0)),
                  pl.BlockSpec((k, m), lambda i: (0, 0))],
        out_specs=pl.BlockSpec((bm, m), lambda i: (i, 0)),
        out_shape=jax.ShapeDtypeStruct((np_rows, m), jnp.float32),
    )(x, w)


def _matmul_split(x, w, np_rows, bm, fw):
    n, k = x.shape
    m = w.shape[1]
    sw = m - fw

    def body(x_ref, w_ref, f_ref, s_ref):
        r = jnp.dot(x_ref[...], w_ref[...], preferred_element_type=jnp.float32)
        f_ref[...] = r[:, :fw].astype(jnp.bfloat16)
        s_ref[...] = r[:, fw:]

    return pl.pallas_call(
        body,
        grid=(np_rows // bm,),
        in_specs=[pl.BlockSpec((bm, k), lambda i: (i, 0)),
                  pl.BlockSpec((k, m), lambda i: (0, 0))],
        out_specs=[pl.BlockSpec((bm, fw), lambda i: (i, 0)),
                   pl.BlockSpec((bm, sw), lambda i: (i, 0))],
        out_shape=[jax.ShapeDtypeStruct((np_rows, fw), jnp.bfloat16),
                   jax.ShapeDtypeStruct((np_rows, sw), jnp.float32)],
    )(x, w)


def _gat_sc_layer_bf16(NP, D, nheads, F, NB):
    npw = NP // _NW
    nchunks = npw // _C
    E = _C * D
    outw = nheads * F
    mesh = plsc.VectorSubcoreMesh(core_axis_name="c", subcore_axis_name="s")

    @functools.partial(
        pl.kernel,
        mesh=mesh,
        compiler_params=pltpu.CompilerParams(use_tc_tiling_on_sc=False,
                                             needs_layout_passes=False),
        out_type=jax.ShapeDtypeStruct((NP, outw), jnp.float32),
        scratch_types=[
            *[pltpu.VMEM((E,), jnp.int32) for _ in range(NB)],
            *[pltpu.VMEM((E, outw), jnp.bfloat16) for _ in range(NB)],
            *[pltpu.VMEM((E, 2 * nheads), jnp.float32) for _ in range(NB)],
            *[pltpu.VMEM((_C, outw), jnp.float32) for _ in range(2)],
            pltpu.VMEM_SHARED((NP, outw), jnp.bfloat16),
            pltpu.VMEM_SHARED((NP, 2 * nheads), jnp.float32),
            *[pltpu.SemaphoreType.DMA for _ in range(3 * NB + 2)],
        ],
    )
    def k(adjf, gf, gst, out, *scr):
        idx_vs = list(scr[0:NB])
        frows_vs = list(scr[NB:2 * NB])
        strows_vs = list(scr[2 * NB:3 * NB])
        o_vs = list(scr[3 * NB:3 * NB + 2])
        gfs = scr[3 * NB + 2]
        gsts = scr[3 * NB + 3]
        sems = scr[3 * NB + 4:]
        fsems = list(sems[0:NB])
        ssems = list(sems[NB:2 * NB])
        isems = list(sems[2 * NB:3 * NB])
        osems = list(sems[3 * NB:3 * NB + 2])
        sid = lax.axis_index("s")
        wid = sid * 2 + lax.axis_index("c")
        base = wid * npw

        rpt = NP // 16
        pltpu.sync_copy(gf.at[pl.ds(sid * rpt, rpt)], gfs.at[pl.ds(sid * rpt, rpt)])
        pltpu.sync_copy(gst.at[pl.ds(sid * rpt, rpt)], gsts.at[pl.ds(sid * rpt, rpt)])
        plsc.subcore_barrier()

        def start_idx(c, b):
            start = (base + c * _C) * D
            pltpu.make_async_copy(adjf.at[pl.ds(start, E)], idx_vs[b],
                                  isems[b]).start()

        def wait_idx(b):
            pltpu.make_async_copy(adjf.at[pl.ds(0, E)], idx_vs[b],
                                  isems[b]).wait()

        def start_gather(b):
            pltpu.make_async_copy(gfs.at[idx_vs[b]], frows_vs[b], fsems[b]).start()
            pltpu.make_async_copy(gsts.at[idx_vs[b]], strows_vs[b], ssems[b]).start()

        def wait_gather(b):
            pltpu.make_async_copy(gfs.at[idx_vs[b]], frows_vs[b], fsems[b]).wait()
            pltpu.make_async_copy(gsts.at[idx_vs[b]], strows_vs[b], ssems[b]).wait()

        def wait_out(ob):
            pltpu.make_async_copy(o_vs[ob], out.at[pl.ds(base, _C)],
                                  osems[ob]).wait()

        def compute(c, frows_v, strows_v, ob):
            nb = base + c * _C
            o_v = o_vs[ob]

            def node_body(i, carry2):
                r0_ = i * D
                it = lax.iota(jnp.int32, 16)
                r0v = jnp.broadcast_to(r0_, (16,)).astype(jnp.int32)
                for h in range(nheads):
                    cs = jnp.full((16,), 2 * h, jnp.int32)
                    s0 = plsc.load_gather(strows_v, [r0_ + it, cs])
                    s1 = plsc.load_gather(strows_v, [r0_ + 16 + it, cs])
                    t = plsc.load_gather(strows_v, [r0v, cs + 1])
                    e0 = s0 + t
                    e1 = s1 + t
                    e0 = jnp.maximum(e0, 0.2 * e0)
                    e1 = jnp.maximum(e1, 0.2 * e1)
                    p0 = jnp.exp(e0)
                    p1 = jnp.exp(e1)
                    z = jnp.sum(p0) + jnp.sum(p1)
                    zrv = 1.0 / jnp.broadcast_to(z, (16,))
                    ae = [jnp.zeros((16,), jnp.float32) for _ in range(4)]
                    ao = [jnp.zeros((16,), jnp.float32) for _ in range(4)]
                    for d in range(D):
                        pd = jnp.broadcast_to((p0 if d < 16 else p1)[d % 16], (16,))
                        row = frows_v[r0_ + d, pl.ds(h * F, 2 * 16)]
                        fe, fo = plsc.unpack(
                            row, format=plsc.PackFormat.INTERLEAVED,
                            preferred_element_type=jnp.float32)
                        ae[d % 4] = ae[d % 4] + pd * fe
                        ao[d % 4] = ao[d % 4] + pd * fo
                    for half, a4 in ((0, ae), (1, ao)):
                        acc = ((a4[0] + a4[1]) + (a4[2] + a4[3])) * zrv
                        acc = jnp.where(acc > 0.0, acc, jnp.exp(acc) - 1.0)
                        o_v[i, pl.ds(h * F + half * 16, 16)] = acc
                return carry2

            lax.fori_loop(0, _C, node_body, 0)
            pltpu.make_async_copy(o_v, out.at[pl.ds(nb, _C)], osems[ob]).start()

        for c in range(NB):
            start_idx(c, c)
        for b in range(NB - 1):
            wait_idx(b)
            start_gather(b)

        def quad_body(q, carry):
            c0 = q * NB
            for b in range(NB):
                c = c0 + b
                wait_gather(b)

                @pl.when(c + NB < nchunks)
                def _():
                    start_idx(c + NB, b)

                @pl.when(c + NB - 1 < nchunks)
                def _():
                    wait_idx((b + NB - 1) % NB)
                    start_gather((b + NB - 1) % NB)

                if b >= 2:
                    wait_out(b % 2)
                else:
                    @pl.when(q > 0)
                    def _():
                        wait_out(b % 2)

                compute(c, frows_vs[b], strows_vs[b], b % 2)

            return carry

        lax.fori_loop(0, nchunks // NB, quad_body, 0)
        wait_out(0)
        wait_out(1)

    return k


def _gat_sc_layer(N, NP, D, nheads, F, GW, apply_elu):
    npw = NP // _NW
    nchunks = npw // _C
    E = _C * D
    outw = nheads * F
    scol0 = nheads * F
    nacc = F // 16
    idx_lim = N * D - E
    mesh = plsc.VectorSubcoreMesh(core_axis_name="c", subcore_axis_name="s")

    @functools.partial(
        pl.kernel,
        mesh=mesh,
        compiler_params=pltpu.CompilerParams(use_tc_tiling_on_sc=False,
                                             needs_layout_passes=False),
        out_type=jax.ShapeDtypeStruct((NP, outw), jnp.float32),
        scratch_types=[
            *[pltpu.VMEM((E,), jnp.int32) for _ in range(_NB)],
            *[pltpu.VMEM((E, GW), jnp.float32) for _ in range(_NB)],
            *[pltpu.VMEM((_C, outw), jnp.float32) for _ in range(2)],
            pltpu.VMEM_SHARED((NP, GW), jnp.float32),
            *[pltpu.SemaphoreType.DMA for _ in range(_NB)],
            *[pltpu.SemaphoreType.DMA for _ in range(_NB)],
            *[pltpu.SemaphoreType.DMA for _ in range(2)],
        ],
    )
    def k(adjf, g, out, *scr):
        idx_vs = list(scr[0:_NB])
        rows_vs = list(scr[_NB:2 * _NB])
        o_vs = list(scr[2 * _NB:2 * _NB + 2])
        gs = scr[2 * _NB + 2]
        gsems = list(scr[2 * _NB + 3:3 * _NB + 3])
        isems = list(scr[3 * _NB + 3:4 * _NB + 3])
        osems = list(scr[4 * _NB + 3:4 * _NB + 5])
        sid = lax.axis_index("s")
        wid = sid * 2 + lax.axis_index("c")
        base = wid * npw

        rpt = NP // 16
        pltpu.sync_copy(g.at[pl.ds(sid * rpt, rpt)], gs.at[pl.ds(sid * rpt, rpt)])
        plsc.subcore_barrier()

        def start_idx(c, b):
            start = (base + c * _C) * D
            pltpu.make_async_copy(adjf.at[pl.ds(start, E)], idx_vs[b],
                                  isems[b]).start()

        def wait_idx(b):
            pltpu.make_async_copy(adjf.at[pl.ds(0, E)], idx_vs[b],
                                  isems[b]).wait()

        def start_gather(b):
            pltpu.make_async_copy(gs.at[idx_vs[b]], rows_vs[b], gsems[b]).start()

        def wait_gather(b):
            pltpu.make_async_copy(gs.at[idx_vs[b]], rows_vs[b], gsems[b]).wait()

        def wait_out(ob):
            pltpu.make_async_copy(o_vs[ob], out.at[pl.ds(base, _C)],
                                  osems[ob]).wait()

        def compute(c, rows_v, ob):
            nb = base + c * _C
            o_v = o_vs[ob]

            def node_body(i, carry2):
                r0_ = i * D
                stv = rows_v[r0_, pl.ds(scol0, 16)]
                for h in range(nheads):
                    scol = scol0 + 2 * h
                    it = lax.iota(jnp.int32, 16)
                    cs = jnp.full((16,), scol, jnp.int32)
                    s0 = plsc.load_gather(rows_v, [r0_ + it, cs])
                    s1 = plsc.load_gather(rows_v, [r0_ + 16 + it, cs])
                    t = jnp.broadcast_to(stv[2 * h + 1], (16,))
                    e0 = s0 + t
                    e1 = s1 + t
                    e0 = jnp.maximum(e0, 0.2 * e0)
                    e1 = jnp.maximum(e1, 0.2 * e1)
                    p0 = jnp.exp(e0)
                    p1 = jnp.exp(e1)
                    z = jnp.sum(p0) + jnp.sum(p1)
                    zrv = 1.0 / jnp.broadcast_to(z, (16,))
                    accs = [[jnp.zeros((16,), jnp.float32) for _ in range(4)]
                            for _ in range(nacc)]
                    for d in range(D):
                        pd = jnp.broadcast_to((p0 if d < 16 else p1)[d % 16], (16,))
                        for fg in range(nacc):
                            col = h * F + fg * 16
                            accs[fg][d % 4] = accs[fg][d % 4] + pd * rows_v[r0_ + d, pl.ds(col, 16)]
                    for fg in range(nacc):
                        a4 = accs[fg]
                        acc = ((a4[0] + a4[1]) + (a4[2] + a4[3])) * zrv
                        if apply_elu:
                            acc = jnp.where(acc > 0.0, acc, jnp.exp(acc) - 1.0)
                        o_v[i, pl.ds(h * F + fg * 16, 16)] = acc
                return carry2

            lax.fori_loop(0, _C, node_body, 0)
            pltpu.make_async_copy(o_v, out.at[pl.ds(nb, _C)], osems[ob]).start()

        for c in range(_NB):
            start_idx(c, c)
        for b in range(_NB - 1):
            wait_idx(b)
            start_gather(b)

        def quad_body(q, carry):
            c0 = q * _NB
            for b in range(_NB):
                c = c0 + b
                wait_gather(b)

                @pl.when(c + _NB < nchunks)
                def _():
                    start_idx(c + _NB, b)

                @pl.when(c + _NB - 1 < nchunks)
                def _():
                    wait_idx((b + _NB - 1) % _NB)
                    start_gather((b + _NB - 1) % _NB)

                if b >= 2:
                    wait_out(b % 2)
                else:
                    @pl.when(q > 0)
                    def _():
                        wait_out(b % 2)

                compute(c, rows_vs[b], b % 2)

            return carry

        lax.fori_loop(0, nchunks // _NB, quad_body, 0)
        wait_out(0)
        wait_out(1)

    return k


def _elu_logsoftmax(zin, n_valid):
    NPl, cls = zin.shape

    def body(z_ref, o_ref):
        zz = z_ref[...]
        x = jnp.where(zz > 0.0, zz, jnp.exp(zz) - 1.0)
        valid = lax.broadcasted_iota(jnp.int32, (NPl, cls), 0) < n_valid
        xm = jnp.where(valid, x, -jnp.inf)
        mx = jnp.max(xm, axis=0, keepdims=True)
        se = jnp.sum(jnp.exp(xm - mx), axis=0, keepdims=True)
        o_ref[...] = x - (mx + jnp.log(se))

    return pl.pallas_call(
        body,
        out_shape=jax.ShapeDtypeStruct((NPl, cls), jnp.float32),
    )(zin)


def kernel(embedding, adj, W_heads, a_heads, W_out, a_out):
    bs, N, nfeat = embedding.shape
    nheads, _, nhid = W_heads.shape
    D = adj.shape[2]
    nclass = W_out.shape[1]
    NP = -(-N // 1024) * 1024

    x = embedding.reshape(N, nfeat)
    xp = jnp.pad(x, ((0, NP - N), (0, 0)))
    adjf = jnp.pad(adj.reshape(N, D), ((0, NP - N), (0, 0))).reshape(NP * D)

    Wc = jnp.swapaxes(W_heads, 0, 1).reshape(nfeat, nheads * nhid)
    a1 = a_heads[:, :nhid, 0]
    a2 = a_heads[:, nhid:, 0]
    sW = jnp.einsum('hfk,hk->fh', W_heads, a1)
    tW = jnp.einsum('hfk,hk->fh', W_heads, a2)
    stW = jnp.stack([sW, tW], axis=2).reshape(nfeat, 2 * nheads)
    M1 = jnp.concatenate([Wc, stW], axis=1)
    Gf, Gst = _matmul_split(xp, M1, NP, 2048, nheads * nhid)

    l1 = _gat_sc_layer_bf16(NP, D, nheads, nhid, 4)
    x1 = l1(adjf, Gf, Gst)

    GW2 = 48
    M2 = jnp.concatenate(
        [W_out, W_out @ a_out[:nclass], W_out @ a_out[nclass:],
         jnp.zeros((nheads * nhid, GW2 - nclass - 2), jnp.float32)],
        axis=1)
    pj = jnp.concatenate([jnp.arange(0, nhid, 2), jnp.arange(1, nhid, 2)])
    row_idx = jnp.concatenate([h * nhid + pj for h in range(nheads)])
    M2p = M2[row_idx]
    G2 = _matmul(x1, M2p, NP, 2048)

    l2 = _gat_sc_layer(N, NP, D, 1, nclass, GW2, False)
    z = l2(adjf, G2)

    out = _elu_logsoftmax(z, N)
    return out[:N].reshape(bs, N, nclass)


# speedup vs baseline: 1.2171x; 1.1041x over previous
import functools

import jax
import jax.numpy as jnp
from jax import lax
from jax.experimental import pallas as pl
from jax.experimental.pallas import tpu as pltpu
from jax.experimental.pallas import tpu_sc as plsc

_NW = 32
_C = 4
_NB = 2


def _matmul(x, w, np_rows, bm):
    n, k = x.shape
    m = w.shape[1]

    def body(x_ref, w_ref, o_ref):
        o_ref[...] = jnp.dot(x_ref[...], w_ref[...],
                             preferred_element_type=jnp.float32)

    return pl.pallas_call(
        body,
        grid=(np_rows // bm,),
        in_specs=[pl.BlockSpec((bm, k), lambda i: (i, 0)),
                  pl.BlockSpec((k, m), lambda i: (0, 0))],
        out_specs=pl.BlockSpec((bm, m), lambda i: (i, 0)),
        out_shape=jax.ShapeDtypeStruct((np_rows, m), jnp.float32),
    )(x, w)


def _gat_sc_layer(N, NP, D, nheads, F, GW, apply_elu):
    npw = NP // _NW
    nchunks = npw // _C
    E = _C * D
    outw = nheads * F
    scol0 = nheads * F
    nacc = F // 16
    idx_lim = N * D - E
    mesh = plsc.VectorSubcoreMesh(core_axis_name="c", subcore_axis_name="s")

    @functools.partial(
        pl.kernel,
        mesh=mesh,
        compiler_params=pltpu.CompilerParams(use_tc_tiling_on_sc=False,
                                             needs_layout_passes=False),
        out_type=jax.ShapeDtypeStruct((NP, outw), jnp.float32),
        scratch_types=[
            *[pltpu.VMEM((E,), jnp.int32) for _ in range(_NB)],
            *[pltpu.VMEM((E, GW), jnp.float32) for _ in range(_NB)],
            *[pltpu.VMEM((_C, outw), jnp.float32) for _ in range(2)],
            pltpu.VMEM_SHARED((NP, GW), jnp.float32),
            *[pltpu.SemaphoreType.DMA for _ in range(_NB)],
            *[pltpu.SemaphoreType.DMA for _ in range(_NB)],
            *[pltpu.SemaphoreType.DMA for _ in range(2)],
        ],
    )
    def k(adjf, g, out, *scr):
        idx_vs = list(scr[0:_NB])
        rows_vs = list(scr[_NB:2 * _NB])
        o_vs = list(scr[2 * _NB:2 * _NB + 2])
        gs = scr[2 * _NB + 2]
        gsems = list(scr[2 * _NB + 3:3 * _NB + 3])
        isems = list(scr[3 * _NB + 3:4 * _NB + 3])
        osems = list(scr[4 * _NB + 3:4 * _NB + 5])
        sid = lax.axis_index("s")
        wid = sid * 2 + lax.axis_index("c")
        base = wid * npw

        rpt = NP // 16
        pltpu.sync_copy(g.at[pl.ds(sid * rpt, rpt)], gs.at[pl.ds(sid * rpt, rpt)])
        plsc.subcore_barrier()

        def start_idx(c, b):
            start = (base + c * _C) * D
            pltpu.make_async_copy(adjf.at[pl.ds(start, E)], idx_vs[b],
                                  isems[b]).start()

        def wait_idx(b):
            pltpu.make_async_copy(adjf.at[pl.ds(0, E)], idx_vs[b],
                                  isems[b]).wait()

        def start_gather(b):
            pltpu.make_async_copy(gs.at[idx_vs[b]], rows_vs[b], gsems[b]).start()

        def wait_gather(b):
            pltpu.make_async_copy(gs.at[idx_vs[b]], rows_vs[b], gsems[b]).wait()

        def wait_out(ob):
            pltpu.make_async_copy(o_vs[ob], out.at[pl.ds(base, _C)],
                                  osems[ob]).wait()

        def compute(c, rows_v, ob):
            nb = base + c * _C
            o_v = o_vs[ob]

            def node_body(i, carry2):
                r0_ = i * D
                stv = rows_v[r0_, pl.ds(scol0, 16)]
                for h in range(nheads):
                    scol = scol0 + 2 * h
                    it = lax.iota(jnp.int32, 16)
                    cs = jnp.full((16,), scol, jnp.int32)
                    s0 = plsc.load_gather(rows_v, [r0_ + it, cs])
                    s1 = plsc.load_gather(rows_v, [r0_ + 16 + it, cs])
                    t = jnp.broadcast_to(stv[2 * h + 1], (16,))
                    e0 = s0 + t
                    e1 = s1 + t
                    e0 = jnp.maximum(e0, 0.2 * e0)
                    e1 = jnp.maximum(e1, 0.2 * e1)
                    p0 = jnp.exp(e0)
                    p1 = jnp.exp(e1)
                    z = jnp.sum(p0) + jnp.sum(p1)
                    zrv = 1.0 / jnp.broadcast_to(z, (16,))
                    accs = [[jnp.zeros((16,), jnp.float32) for _ in range(4)]
                            for _ in range(nacc)]
                    for d in range(D):
                        pd = jnp.broadcast_to((p0 if d < 16 else p1)[d % 16], (16,))
                        for fg in range(nacc):
                            col = h * F + fg * 16
                            accs[fg][d % 4] = accs[fg][d % 4] + pd * rows_v[r0_ + d, pl.ds(col, 16)]
                    for fg in range(nacc):
                        a4 = accs[fg]
                        acc = ((a4[0] + a4[1]) + (a4[2] + a4[3])) * zrv
                        if apply_elu:
                            acc = jnp.where(acc > 0.0, acc, jnp.exp(acc) - 1.0)
                        o_v[i, pl.ds(h * F + fg * 16, 16)] = acc
                return carry2

            lax.fori_loop(0, _C, node_body, 0)
            pltpu.make_async_copy(o_v, out.at[pl.ds(nb, _C)], osems[ob]).start()

        for c in range(_NB):
            start_idx(c, c)
        for b in range(_NB - 1):
            wait_idx(b)
            start_gather(b)

        def quad_body(q, carry):
            c0 = q * _NB
            for b in range(_NB):
                c = c0 + b
                wait_gather(b)

                @pl.when(c + _NB < nchunks)
                def _():
                    start_idx(c + _NB, b)

                @pl.when(c + _NB - 1 < nchunks)
                def _():
                    wait_idx((b + _NB - 1) % _NB)
                    start_gather((b + _NB - 1) % _NB)

                if b >= 2:
                    wait_out(b % 2)
                else:
                    @pl.when(q > 0)
                    def _():
                        wait_out(b % 2)

                compute(c, rows_vs[b], b % 2)

            return carry

        lax.fori_loop(0, nchunks // _NB, quad_body, 0)
        wait_out(0)
        wait_out(1)

    return k


def _elu_logsoftmax(zin, n_valid):
    NPl, cls = zin.shape

    def body(z_ref, o_ref):
        zz = z_ref[...]
        x = jnp.where(zz > 0.0, zz, jnp.exp(zz) - 1.0)
        valid = lax.broadcasted_iota(jnp.int32, (NPl, cls), 0) < n_valid
        xm = jnp.where(valid, x, -jnp.inf)
        mx = jnp.max(xm, axis=0, keepdims=True)
        se = jnp.sum(jnp.exp(xm - mx), axis=0, keepdims=True)
        o_ref[...] = x - (mx + jnp.log(se))

    return pl.pallas_call(
        body,
        out_shape=jax.ShapeDtypeStruct((NPl, cls), jnp.float32),
    )(zin)


def kernel(embedding, adj, W_heads, a_heads, W_out, a_out):
    bs, N, nfeat = embedding.shape
    nheads, _, nhid = W_heads.shape
    D = adj.shape[2]
    nclass = W_out.shape[1]
    NP = -(-N // 1024) * 1024

    x = embedding.reshape(N, nfeat)
    xp = jnp.pad(x, ((0, NP - N), (0, 0)))
    adjf = jnp.pad(adj.reshape(N, D), ((0, NP - N), (0, 0))).reshape(NP * D)

    Wc = jnp.swapaxes(W_heads, 0, 1).reshape(nfeat, nheads * nhid)
    a1 = a_heads[:, :nhid, 0]
    a2 = a_heads[:, nhid:, 0]
    sW = jnp.einsum('hfk,hk->fh', W_heads, a1)
    tW = jnp.einsum('hfk,hk->fh', W_heads, a2)
    stW = jnp.stack([sW, tW], axis=2).reshape(nfeat, 2 * nheads)
    GW1 = 144
    M1 = jnp.concatenate(
        [Wc, stW,
         jnp.zeros((nfeat, GW1 - nheads * nhid - 2 * nheads), jnp.float32)],
        axis=1)
    G1 = _matmul(xp, M1, NP, 2048)

    l1 = _gat_sc_layer(N, NP, D, nheads, nhid, GW1, True)
    x1 = l1(adjf, G1)

    GW2 = 48
    M2 = jnp.concatenate(
        [W_out, W_out @ a_out[:nclass], W_out @ a_out[nclass:],
         jnp.zeros((nheads * nhid, GW2 - nclass - 2), jnp.float32)],
        axis=1)
    G2 = _matmul(x1, M2, NP, 2048)

    l2 = _gat_sc_layer(N, NP, D, 1, nclass, GW2, False)
    z = l2(adjf, G2)

    out = _elu_logsoftmax(z, N)
    return out[:N].reshape(bs, N, nclass)


# speedup vs baseline: 1.2317x; 1.0120x over previous
import functools

import jax
import jax.numpy as jnp
from jax import lax
from jax.experimental import pallas as pl
from jax.experimental.pallas import tpu as pltpu
from jax.experimental.pallas import tpu_sc as plsc

_NW = 32
_C = 4
def _matmul(x, w, np_rows, bm):
    n, k = x.shape
    m = w.shape[1]

    def body(x_ref, w_ref, o_ref):
        o_ref[...] = jnp.dot(x_ref[...], w_ref[...],
                             preferred_element_type=jnp.float32)

    return pl.pallas_call(
        body,
        grid=(np_rows // bm,),
        in_specs=[pl.BlockSpec((bm, k), lambda i: (i, 0)),
                  pl.BlockSpec((k, m), lambda i: (0, 0))],
        out_specs=pl.BlockSpec((bm, m), lambda i: (i, 0)),
        out_shape=jax.ShapeDtypeStruct((np_rows, m), jnp.float32),
    )(x, w)


def _gat_sc_layer(N, NP, D, nheads, F, GW, apply_elu, _NB):
    npw = NP // _NW
    nchunks = npw // _C
    E = _C * D
    outw = nheads * F
    scol0 = nheads * F
    nacc = F // 16
    idx_lim = N * D - E
    mesh = plsc.VectorSubcoreMesh(core_axis_name="c", subcore_axis_name="s")

    @functools.partial(
        pl.kernel,
        mesh=mesh,
        compiler_params=pltpu.CompilerParams(use_tc_tiling_on_sc=False,
                                             needs_layout_passes=False),
        out_type=jax.ShapeDtypeStruct((NP, outw), jnp.float32),
        scratch_types=[
            *[pltpu.VMEM((E,), jnp.int32) for _ in range(_NB)],
            *[pltpu.VMEM((E, GW), jnp.float32) for _ in range(_NB)],
            *[pltpu.VMEM((_C, outw), jnp.float32) for _ in range(2)],
            pltpu.VMEM_SHARED((NP, GW), jnp.float32),
            *[pltpu.SemaphoreType.DMA for _ in range(_NB)],
            *[pltpu.SemaphoreType.DMA for _ in range(_NB)],
            *[pltpu.SemaphoreType.DMA for _ in range(2)],
        ],
    )
    def k(adjf, g, out, *scr):
        idx_vs = list(scr[0:_NB])
        rows_vs = list(scr[_NB:2 * _NB])
        o_vs = list(scr[2 * _NB:2 * _NB + 2])
        gs = scr[2 * _NB + 2]
        gsems = list(scr[2 * _NB + 3:3 * _NB + 3])
        isems = list(scr[3 * _NB + 3:4 * _NB + 3])
        osems = list(scr[4 * _NB + 3:4 * _NB + 5])
        sid = lax.axis_index("s")
        wid = sid * 2 + lax.axis_index("c")
        base = wid * npw

        rpt = NP // 16
        pltpu.sync_copy(g.at[pl.ds(sid * rpt, rpt)], gs.at[pl.ds(sid * rpt, rpt)])
        plsc.subcore_barrier()

        def start_idx(c, b):
            start = (base + c * _C) * D
            pltpu.make_async_copy(adjf.at[pl.ds(start, E)], idx_vs[b],
                                  isems[b]).start()

        def wait_idx(b):
            pltpu.make_async_copy(adjf.at[pl.ds(0, E)], idx_vs[b],
                                  isems[b]).wait()

        def start_gather(b):
            pltpu.make_async_copy(gs.at[idx_vs[b]], rows_vs[b], gsems[b]).start()

        def wait_gather(b):
            pltpu.make_async_copy(gs.at[idx_vs[b]], rows_vs[b], gsems[b]).wait()

        def wait_out(ob):
            pltpu.make_async_copy(o_vs[ob], out.at[pl.ds(base, _C)],
                                  osems[ob]).wait()

        def compute(c, rows_v, ob):
            nb = base + c * _C
            o_v = o_vs[ob]

            def node_body(i, carry2):
                r0_ = i * D
                stv = rows_v[r0_, pl.ds(scol0, 16)]
                for h in range(nheads):
                    scol = scol0 + 2 * h
                    it = lax.iota(jnp.int32, 16)
                    cs = jnp.full((16,), scol, jnp.int32)
                    s0 = plsc.load_gather(rows_v, [r0_ + it, cs])
                    s1 = plsc.load_gather(rows_v, [r0_ + 16 + it, cs])
                    t = jnp.broadcast_to(stv[2 * h + 1], (16,))
                    e0 = s0 + t
                    e1 = s1 + t
                    e0 = jnp.maximum(e0, 0.2 * e0)
                    e1 = jnp.maximum(e1, 0.2 * e1)
                    p0 = jnp.exp(e0)
                    p1 = jnp.exp(e1)
                    z = jnp.sum(p0) + jnp.sum(p1)
                    zrv = 1.0 / jnp.broadcast_to(z, (16,))
                    accs = [[jnp.zeros((16,), jnp.float32) for _ in range(4)]
                            for _ in range(nacc)]
                    for d in range(D):
                        pd = jnp.broadcast_to((p0 if d < 16 else p1)[d % 16], (16,))
                        for fg in range(nacc):
                            col = h * F + fg * 16
                            accs[fg][d % 4] = accs[fg][d % 4] + pd * rows_v[r0_ + d, pl.ds(col, 16)]
                    for fg in range(nacc):
                        a4 = accs[fg]
                        acc = ((a4[0] + a4[1]) + (a4[2] + a4[3])) * zrv
                        if apply_elu:
                            acc = jnp.where(acc > 0.0, acc, jnp.exp(acc) - 1.0)
                        o_v[i, pl.ds(h * F + fg * 16, 16)] = acc
                return carry2

            lax.fori_loop(0, _C, node_body, 0)
            pltpu.make_async_copy(o_v, out.at[pl.ds(nb, _C)], osems[ob]).start()

        for c in range(_NB):
            start_idx(c, c)
        for b in range(_NB - 1):
            wait_idx(b)
            start_gather(b)

        def quad_body(q, carry):
            c0 = q * _NB
            for b in range(_NB):
                c = c0 + b
                wait_gather(b)

                @pl.when(c + _NB < nchunks)
                def _():
                    start_idx(c + _NB, b)

                @pl.when(c + _NB - 1 < nchunks)
                def _():
                    wait_idx((b + _NB - 1) % _NB)
                    start_gather((b + _NB - 1) % _NB)

                if b >= 2:
                    wait_out(b % 2)
                else:
                    @pl.when(q > 0)
                    def _():
                        wait_out(b % 2)

                compute(c, rows_vs[b], b % 2)

            return carry

        lax.fori_loop(0, nchunks // _NB, quad_body, 0)
        wait_out(0)
        wait_out(1)

    return k


def _elu_logsoftmax(zin, n_valid):
    NPl, cls = zin.shape

    def body(z_ref, o_ref):
        zz = z_ref[...]
        x = jnp.where(zz > 0.0, zz, jnp.exp(zz) - 1.0)
        valid = lax.broadcasted_iota(jnp.int32, (NPl, cls), 0) < n_valid
        xm = jnp.where(valid, x, -jnp.inf)
        mx = jnp.max(xm, axis=0, keepdims=True)
        se = jnp.sum(jnp.exp(xm - mx), axis=0, keepdims=True)
        o_ref[...] = x - (mx + jnp.log(se))

    return pl.pallas_call(
        body,
        out_shape=jax.ShapeDtypeStruct((NPl, cls), jnp.float32),
    )(zin)


def kernel(embedding, adj, W_heads, a_heads, W_out, a_out):
    bs, N, nfeat = embedding.shape
    nheads, _, nhid = W_heads.shape
    D = adj.shape[2]
    nclass = W_out.shape[1]
    NP = -(-N // 1024) * 1024

    x = embedding.reshape(N, nfeat)
    xp = jnp.pad(x, ((0, NP - N), (0, 0)))
    adjf = jnp.pad(adj.reshape(N, D), ((0, NP - N), (0, 0))).reshape(NP * D)

    Wc = jnp.swapaxes(W_heads, 0, 1).reshape(nfeat, nheads * nhid)
    a1 = a_heads[:, :nhid, 0]
    a2 = a_heads[:, nhid:, 0]
    sW = jnp.einsum('hfk,hk->fh', W_heads, a1)
    tW = jnp.einsum('hfk,hk->fh', W_heads, a2)
    stW = jnp.stack([sW, tW], axis=2).reshape(nfeat, 2 * nheads)
    GW1 = 144
    M1 = jnp.concatenate(
        [Wc, stW,
         jnp.zeros((nfeat, GW1 - nheads * nhid - 2 * nheads), jnp.float32)],
        axis=1)
    G1 = _matmul(xp, M1, NP, 2048)

    l1 = _gat_sc_layer(N, NP, D, nheads, nhid, GW1, True, 2)
    x1 = l1(adjf, G1)

    GW2 = 48
    M2 = jnp.concatenate(
        [W_out, W_out @ a_out[:nclass], W_out @ a_out[nclass:],
         jnp.zeros((nheads * nhid, GW2 - nclass - 2), jnp.float32)],
        axis=1)
    G2 = _matmul(x1, M2, NP, 2048)

    l2 = _gat_sc_layer(N, NP, D, 1, nclass, GW2, False, 4)
    z = l2(adjf, G2)

    out = _elu_logsoftmax(z, N)
    return out[:N].reshape(bs, N, nclass)


# speedup vs baseline: 1.2702x; 1.0312x over previous
import functools

import jax
import jax.numpy as jnp
from jax import lax
from jax.experimental import pallas as pl
from jax.experimental.pallas import tpu as pltpu
from jax.experimental.pallas import tpu_sc as plsc

_NW = 32
_C = 4
def _matmul(x, w, np_rows, bm):
    n, k = x.shape
    m = w.shape[1]

    def body(x_ref, w_ref, o_ref):
        o_ref[...] = jnp.dot(x_ref[...], w_ref[...],
                             preferred_element_type=jnp.float32)

    return pl.pallas_call(
        body,
        grid=(np_rows // bm,),
        in_specs=[pl.BlockSpec((bm, k), lambda i: (i, 0)),
                  pl.BlockSpec((k, m), lambda i: (0, 0))],
        out_specs=pl.BlockSpec((bm, m), lambda i: (i, 0)),
        out_shape=jax.ShapeDtypeStruct((np_rows, m), jnp.float32),
    )(x, w)


def _gat_sc_layer(N, NP, D, nheads, F, GW, apply_elu, _NB):
    npw = NP // _NW
    nchunks = npw // _C
    E = _C * D
    outw = nheads * F
    scol0 = nheads * F
    nacc = F // 16
    idx_lim = N * D - E
    mesh = plsc.VectorSubcoreMesh(core_axis_name="c", subcore_axis_name="s")

    @functools.partial(
        pl.kernel,
        mesh=mesh,
        compiler_params=pltpu.CompilerParams(use_tc_tiling_on_sc=False,
                                             needs_layout_passes=False),
        out_type=jax.ShapeDtypeStruct((NP, outw), jnp.float32),
        scratch_types=[
            *[pltpu.VMEM((E,), jnp.int32) for _ in range(_NB)],
            *[pltpu.VMEM((E, GW), jnp.float32) for _ in range(_NB)],
            *[pltpu.VMEM((_C, outw), jnp.float32) for _ in range(2)],
            pltpu.VMEM_SHARED((NP, GW), jnp.float32),
            *[pltpu.SemaphoreType.DMA for _ in range(_NB)],
            *[pltpu.SemaphoreType.DMA for _ in range(_NB)],
            *[pltpu.SemaphoreType.DMA for _ in range(2)],
        ],
    )
    def k(adjf, g, out, *scr):
        idx_vs = list(scr[0:_NB])
        rows_vs = list(scr[_NB:2 * _NB])
        o_vs = list(scr[2 * _NB:2 * _NB + 2])
        gs = scr[2 * _NB + 2]
        gsems = list(scr[2 * _NB + 3:3 * _NB + 3])
        isems = list(scr[3 * _NB + 3:4 * _NB + 3])
        osems = list(scr[4 * _NB + 3:4 * _NB + 5])
        sid = lax.axis_index("s")
        wid = sid * 2 + lax.axis_index("c")
        base = wid * npw

        rpt = NP // 16
        pltpu.sync_copy(g.at[pl.ds(sid * rpt, rpt)], gs.at[pl.ds(sid * rpt, rpt)])
        plsc.subcore_barrier()

        def start_idx(c, b):
            start = (base + c * _C) * D
            pltpu.make_async_copy(adjf.at[pl.ds(start, E)], idx_vs[b],
                                  isems[b]).start()

        def wait_idx(b):
            pltpu.make_async_copy(adjf.at[pl.ds(0, E)], idx_vs[b],
                                  isems[b]).wait()

        def start_gather(b):
            pltpu.make_async_copy(gs.at[idx_vs[b]], rows_vs[b], gsems[b]).start()

        def wait_gather(b):
            pltpu.make_async_copy(gs.at[idx_vs[b]], rows_vs[b], gsems[b]).wait()

        def wait_out(ob):
            pltpu.make_async_copy(o_vs[ob], out.at[pl.ds(base, _C)],
                                  osems[ob]).wait()

        def compute(c, rows_v, ob):
            nb = base + c * _C
            o_v = o_vs[ob]

            def node_body(i, carry2):
                r0_ = i * D
                stv = rows_v[r0_, pl.ds(scol0, 16)]
                for h in range(nheads):
                    scol = scol0 + 2 * h
                    it = lax.iota(jnp.int32, 16)
                    cs = jnp.full((16,), scol, jnp.int32)
                    s0 = plsc.load_gather(rows_v, [r0_ + it, cs])
                    s1 = plsc.load_gather(rows_v, [r0_ + 16 + it, cs])
                    t = jnp.broadcast_to(stv[2 * h + 1], (16,))
                    e0 = s0 + t
                    e1 = s1 + t
                    e0 = jnp.maximum(e0, 0.2 * e0)
                    e1 = jnp.maximum(e1, 0.2 * e1)
                    p0 = jnp.exp(e0)
                    p1 = jnp.exp(e1)
                    z = jnp.sum(p0) + jnp.sum(p1)
                    zrv = 1.0 / jnp.broadcast_to(z, (16,))
                    accs = [[jnp.zeros((16,), jnp.float32) for _ in range(4)]
                            for _ in range(nacc)]
                    for d in range(D):
                        pd = jnp.broadcast_to((p0 if d < 16 else p1)[d % 16], (16,))
                        for fg in range(nacc):
                            col = h * F + fg * 16
                            accs[fg][d % 4] = accs[fg][d % 4] + pd * rows_v[r0_ + d, pl.ds(col, 16)]
                    for fg in range(nacc):
                        a4 = accs[fg]
                        acc = ((a4[0] + a4[1]) + (a4[2] + a4[3])) * zrv
                        if apply_elu:
                            acc = jnp.where(acc > 0.0, acc, jnp.exp(acc) - 1.0)
                        o_v[i, pl.ds(h * F + fg * 16, 16)] = acc
                return carry2

            lax.fori_loop(0, _C, node_body, 0)
            pltpu.make_async_copy(o_v, out.at[pl.ds(nb, _C)], osems[ob]).start()

        for c in range(_NB):
            start_idx(c, c)
        for b in range(_NB - 1):
            wait_idx(b)
            start_gather(b)

        def quad_body(q, carry):
            c0 = q * _NB
            for b in range(_NB):
                c = c0 + b
                wait_gather(b)

                @pl.when(c + _NB < nchunks)
                def _():
                    start_idx(c + _NB, b)

                @pl.when(c + _NB - 1 < nchunks)
                def _():
                    wait_idx((b + _NB - 1) % _NB)
                    start_gather((b + _NB - 1) % _NB)

                if b >= 2:
                    wait_out(b % 2)
                else:
                    @pl.when(q > 0)
                    def _():
                        wait_out(b % 2)

                compute(c, rows_vs[b], b % 2)

            return carry

        lax.fori_loop(0, nchunks // _NB, quad_body, 0)
        wait_out(0)
        wait_out(1)

    return k


def _elu_logsoftmax(zin, n_valid):
    NPl, cls = zin.shape
    g = 128 // cls
    R = NPl // g
    z2 = zin.reshape(R, 128)

    def body(z_ref, o_ref):
        zz = z_ref[...]
        x = jnp.where(zz > 0.0, zz, jnp.exp(zz) - 1.0)
        row = lax.broadcasted_iota(jnp.int32, (R, 128), 0)
        lane = lax.broadcasted_iota(jnp.int32, (R, 128), 1)
        node = row * g + lane // cls
        xm = jnp.where(node < n_valid, x, -jnp.inf)
        m128 = jnp.max(xm, axis=0, keepdims=True)
        m32 = m128[:, 0:cls]
        for k in range(1, g):
            m32 = jnp.maximum(m32, m128[:, k * cls:(k + 1) * cls])
        mb = jnp.concatenate([m32] * g, axis=1)
        se128 = jnp.sum(jnp.exp(xm - mb), axis=0, keepdims=True)
        se32 = se128[:, 0:cls]
        for k in range(1, g):
            se32 = se32 + se128[:, k * cls:(k + 1) * cls]
        lse = m32 + jnp.log(se32)
        lb = jnp.concatenate([lse] * g, axis=1)
        o_ref[...] = x - lb

    out2 = pl.pallas_call(
        body,
        out_shape=jax.ShapeDtypeStruct((R, 128), jnp.float32),
    )(z2)
    return out2.reshape(NPl, cls)


def kernel(embedding, adj, W_heads, a_heads, W_out, a_out):
    bs, N, nfeat = embedding.shape
    nheads, _, nhid = W_heads.shape
    D = adj.shape[2]
    nclass = W_out.shape[1]
    NP = -(-N // 1024) * 1024

    x = embedding.reshape(N, nfeat)
    xp = jnp.pad(x, ((0, NP - N), (0, 0)))
    adjf = jnp.pad(adj.reshape(N, D), ((0, NP - N), (0, 0))).reshape(NP * D)

    Wc = jnp.swapaxes(W_heads, 0, 1).reshape(nfeat, nheads * nhid)
    a1 = a_heads[:, :nhid, 0]
    a2 = a_heads[:, nhid:, 0]
    sW = jnp.einsum('hfk,hk->fh', W_heads, a1)
    tW = jnp.einsum('hfk,hk->fh', W_heads, a2)
    stW = jnp.stack([sW, tW], axis=2).reshape(nfeat, 2 * nheads)
    GW1 = 144
    M1 = jnp.concatenate(
        [Wc, stW,
         jnp.zeros((nfeat, GW1 - nheads * nhid - 2 * nheads), jnp.float32)],
        axis=1)
    G1 = _matmul(xp, M1, NP, 2048)

    l1 = _gat_sc_layer(N, NP, D, nheads, nhid, GW1, True, 2)
    x1 = l1(adjf, G1)

    GW2 = 48
    M2 = jnp.concatenate(
        [W_out, W_out @ a_out[:nclass], W_out @ a_out[nclass:],
         jnp.zeros((nheads * nhid, GW2 - nclass - 2), jnp.float32)],
        axis=1)
    G2 = _matmul(x1, M2, NP, 2048)

    l2 = _gat_sc_layer(N, NP, D, 1, nclass, GW2, False, 4)
    z = l2(adjf, G2)

    out = _elu_logsoftmax(z, N)
    return out[:N].reshape(bs, N, nclass)
